# baseline (speedup 1.0000x reference)
import functools

import jax
import jax.numpy as jnp
from jax import lax
from jax.experimental import pallas as pl
from jax.experimental.pallas import tpu as pltpu
from jax.experimental.pallas import tpu_sc as plsc

VOCAB = 1000
EMBED = 32
BATCH = 4096
HIST = 200

N = BATCH * HIST
NWORKERS = 32
IPW = N // NWORKERS
CHI = 256
NCHUNK = IPW // CHI
NS = 4
CHF = CHI * EMBED

_mesh = plsc.VectorSubcoreMesh(core_axis_name="c", subcore_axis_name="s")


@functools.partial(
    pl.kernel,
    mesh=_mesh,
    out_type=jax.ShapeDtypeStruct((N, EMBED), jnp.float32),
    scratch_types=[
        pltpu.VMEM((NS, CHI * 16), jnp.int32),
        pltpu.VMEM((VOCAB * EMBED,), jnp.float32),
        pltpu.VMEM((NS, CHI, EMBED), jnp.float32),
        pltpu.SemaphoreType.DMA((NS,)),
        pltpu.SemaphoreType.DMA((NS,)),
    ],
    compiler_params=pltpu.CompilerParams(
        use_tc_tiling_on_sc=False, needs_layout_passes=False
    ),
)
def _gather_kernel(idx_hbm, table_hbm, out_hbm, idxr, table_v, ring,
                   out_sems, idx_sems):
    wid = lax.axis_index("s") * 2 + lax.axis_index("c")
    ibase = wid * IPW
    pltpu.sync_copy(table_hbm, table_v)

    lane = lax.iota(jnp.int32, 16)
    CHE = CHI * 16

    def fire_idx(j):
        s = j % NS
        pltpu.async_copy(
            idx_hbm.at[pl.ds((ibase + j * CHI) * 16, CHE)],
            idxr.at[s],
            idx_sems.at[s],
        )

    fire_idx(0)

    def chunk_body(j, carry):
        s = j % NS

        @pl.when(j + 1 < NCHUNK)
        def _prefetch_idx():
            fire_idx(j + 1)

        pltpu.make_async_copy(
            idx_hbm.at[pl.ds(0, CHE)], idxr.at[s], idx_sems.at[s]
        ).wait()

        @pl.when(j >= NS)
        def _slot_free():
            pltpu.make_async_copy(
                ring.at[s], out_hbm.at[pl.ds(0, CHI)], out_sems.at[s]
            ).wait()

        for p in range(CHI):
            spl = idxr[s, pl.ds(p * 16, 16)]
            addr0 = spl * EMBED + lane
            v0 = plsc.load_gather(table_v, [addr0])
            v1 = plsc.load_gather(table_v, [addr0 + 16])
            ring[s, p, pl.ds(0, 16)] = v0
            ring[s, p, pl.ds(16, 16)] = v1

        pltpu.async_copy(
            ring.at[s],
            out_hbm.at[pl.ds(ibase + j * CHI, CHI)],
            out_sems.at[s],
        )
        return carry

    lax.fori_loop(0, NCHUNK, chunk_body, 0)
    for s in range(NS):
        pltpu.make_async_copy(
            ring.at[s], out_hbm.at[pl.ds(0, CHI)], out_sems.at[s]
        ).wait()


def kernel(word_idx_list, W):
    idx = jnp.repeat(word_idx_list.astype(jnp.int32).reshape(N), 16)
    out = _gather_kernel(idx, W.reshape(VOCAB * EMBED))
    return out.reshape(BATCH, HIST, EMBED)


# speedup vs baseline: 2.4909x; 2.4909x over previous
import functools

import jax
import jax.numpy as jnp
from jax import lax
from jax.experimental import pallas as pl
from jax.experimental.pallas import tpu as pltpu
from jax.experimental.pallas import tpu_sc as plsc

VOCAB = 1000
EMBED = 32
BATCH = 4096
HIST = 200

LANE = 128
N = BATCH * HIST
ROWS = N // LANE
NWORKERS = 32
RPW = ROWS // NWORKERS
IPW = RPW * LANE

SCH = 4
CCH = 2
NIT_S = 33
NIT_C = 34
CROW0 = NIT_S * SCH
NIT = 34

_mesh = plsc.VectorSubcoreMesh(core_axis_name="c", subcore_axis_name="s")


@functools.partial(
    pl.kernel,
    mesh=_mesh,
    out_type=jax.ShapeDtypeStruct((N, EMBED), jnp.float32),
    scratch_types=[
        pltpu.VMEM((RPW, LANE), jnp.int32),
        pltpu.VMEM((VOCAB, EMBED), jnp.float32),
        pltpu.VMEM_SHARED((VOCAB, EMBED), jnp.float32),
        pltpu.VMEM((2, SCH * LANE, EMBED), jnp.float32),
        pltpu.VMEM((2, CCH * LANE, EMBED), jnp.float32),
        pltpu.SemaphoreType.DMA((2,)),
        pltpu.SemaphoreType.DMA((2,)),
        pltpu.SemaphoreType.DMA((2,)),
    ],
    compiler_params=pltpu.CompilerParams(
        use_tc_tiling_on_sc=False, needs_layout_passes=False
    ),
)
def _gather_kernel(idx_hbm, table_hbm, out_hbm, idx_v, table_v,
                   table_sh, sring, cring, gat_sems, sout_sems, cout_sems):
    sid = lax.axis_index("s")
    wid = sid * 2 + lax.axis_index("c")
    ibase = wid * IPW
    rbase = wid * RPW

    @pl.when(sid == 0)
    def _stage_spmem_table():
        pltpu.sync_copy(table_hbm, table_sh)

    pltpu.sync_copy(table_hbm, table_v)
    pltpu.sync_copy(idx_hbm.at[pl.ds(rbase, RPW)], idx_v)
    plsc.subcore_barrier()

    lane16 = lax.iota(jnp.int32, 16)

    def body(t, carry):
        ss = t % 2
        cs = t % 2

        @pl.when(t < NIT_S)
        def _stream_fire():
            @pl.when(t >= 2)
            def _slot_free():
                pltpu.make_async_copy(
                    sring.at[ss], out_hbm.at[pl.ds(0, SCH * LANE)],
                    sout_sems.at[ss],
                ).wait()

            for k in range(SCH):
                pltpu.async_copy(
                    table_sh.at[idx_v.at[t * SCH + k]],
                    sring.at[ss].at[pl.ds(k * LANE, LANE)],
                    gat_sems.at[ss],
                )

        @pl.when(t < NIT_C)
        def _compute():
            @pl.when(t >= 2)
            def _slot_free():
                pltpu.make_async_copy(
                    cring.at[cs], out_hbm.at[pl.ds(0, CCH * LANE)],
                    cout_sems.at[cs],
                ).wait()

            for r in range(CCH):
                row = CROW0 + t * CCH + r
                for p in range(LANE):
                    spl = plsc.load_gather(
                        idx_v,
                        [jnp.full((16,), row, jnp.int32),
                         jnp.full((16,), p, jnp.int32)],
                    )
                    v0 = plsc.load_gather(table_v, [spl, lane16])
                    v1 = plsc.load_gather(table_v, [spl, lane16 + 16])
                    cring[cs, r * LANE + p, pl.ds(0, 16)] = v0
                    cring[cs, r * LANE + p, pl.ds(16, 16)] = v1

            pltpu.async_copy(
                cring.at[cs],
                out_hbm.at[pl.ds(ibase + (CROW0 + t * CCH) * LANE, CCH * LANE)],
                cout_sems.at[cs],
            )

        @pl.when(t < NIT_S)
        def _stream_drain():
            for _ in range(SCH):
                pltpu.make_async_copy(
                    table_sh.at[idx_v.at[0]],
                    sring.at[ss].at[pl.ds(0, LANE)],
                    gat_sems.at[ss],
                ).wait()
            pltpu.async_copy(
                sring.at[ss],
                out_hbm.at[pl.ds(ibase + t * SCH * LANE, SCH * LANE)],
                sout_sems.at[ss],
            )

        return carry

    lax.fori_loop(0, NIT, body, 0)

    for s in range(2):
        pltpu.make_async_copy(
            sring.at[s], out_hbm.at[pl.ds(0, SCH * LANE)], sout_sems.at[s]
        ).wait()
        pltpu.make_async_copy(
            cring.at[s], out_hbm.at[pl.ds(0, CCH * LANE)], cout_sems.at[s]
        ).wait()


def kernel(word_idx_list, W):
    idx = word_idx_list.astype(jnp.int32).reshape(ROWS, LANE)
    out = _gather_kernel(idx, W)
    return out.reshape(BATCH, HIST, EMBED)
